# Initial kernel scaffold; baseline (speedup 1.0000x reference)
import jax
import jax.numpy as jnp
from jax.experimental import pallas as pl


def kernel(features_of_nodes, node_type_ids, node_ids_for_edges, features_of_edges, edge_type_ids, global_features, batch_ids, eW1, eb1, eW2, eb2, n1W1, n1b1, n1W2, n1b2, n2W1, n2b1, n2W2, n2b2, gW1, gb1, gW2, gb2):
    raise NotImplementedError("write your pallas kernel here")


# speedup vs baseline: 2.9879x; 2.9879x over previous
import functools

import jax
import jax.numpy as jnp
from jax import lax
from jax.experimental import pallas as pl
from jax.experimental.pallas import tpu as pltpu
from jax.experimental.pallas import tpu_sc as plsc

_N = 10000
_E = 320000
_B = 16
_DF = 128
_DE = 16
_DU = 32
_H = 32

_NC = 2
_NS = 16
_NW = _NC * _NS
_C = 80
_PER_W = _E // _NW
_NCHUNK = _PER_W // _C

_BN = 2000
_BE = 2000

_f32 = jnp.float32


def _dot(a, b):
    return lax.dot_general(a, b, (((1,), (0,)), ((), ())),
                           preferred_element_type=_f32,
                           precision=lax.Precision.HIGHEST)


def _dot_t(a, b):
    return lax.dot_general(a, b, (((0,), (0,)), ((), ())),
                           preferred_element_type=_f32,
                           precision=lax.Precision.HIGHEST)


def _pre_body(x_ref, b_ref, u_ref, eW1xs_ref, eW1xd_ref, eW1u_ref, n1W1x_ref,
              n2W1x_ref, n2W1u_ref, eb1_ref, n1b1_ref, n2b1_ref,
              tsrc_ref, tdst_ref, xp2_ref):
    xb = x_ref[...]
    bid = b_ref[...]
    iota = lax.broadcasted_iota(jnp.int32, (_BN, _B), 1)
    oh = (bid == iota).astype(_f32)
    ue = _dot(u_ref[...], eW1u_ref[...])
    un = _dot(u_ref[...], n2W1u_ref[...])
    tsrc_ref[...] = _dot(xb, eW1xs_ref[...]) + _dot(oh, ue) + eb1_ref[...]
    tdst_ref[...] = jnp.concatenate(
        [_dot(xb, eW1xd_ref[...]),
         _dot(xb, n1W1x_ref[...]) + n1b1_ref[...]], axis=1)
    xp2_ref[...] = _dot(xb, n2W1x_ref[...]) + _dot(oh, un) + n2b1_ref[...]


def _precompute(x, bids2, u, eW1xs, eW1xd, eW1u, n1W1x, n2W1x, n2W1u,
                eb1, n1b1, n2b1):
    full = lambda shp: pl.BlockSpec(shp, lambda i: (0,) * len(shp))
    return pl.pallas_call(
        _pre_body,
        grid=(_N // _BN,),
        in_specs=[
            pl.BlockSpec((_BN, _DF), lambda i: (i, 0)),
            pl.BlockSpec((_BN, 1), lambda i: (i, 0)),
            full((_B, _DU)),
            full((_DF, _H)), full((_DF, _H)), full((_DU, _H)),
            full((_DF, _H)), full((_DF, _H)), full((_DU, _H)),
            full((1, _H)), full((1, _H)), full((1, _H)),
        ],
        out_specs=[
            pl.BlockSpec((_BN, _H), lambda i: (i, 0)),
            pl.BlockSpec((_BN, 2 * _H), lambda i: (i, 0)),
            pl.BlockSpec((_BN, _H), lambda i: (i, 0)),
        ],
        out_shape=[
            jax.ShapeDtypeStruct((_N, _H), _f32),
            jax.ShapeDtypeStruct((_N, 2 * _H), _f32),
            jax.ShapeDtypeStruct((_N, _H), _f32),
        ],
    )(x, bids2, u, eW1xs, eW1xd, eW1u, n1W1x, n2W1x, n2W1u, eb1, n1b1, n2b1)


def _sc_mesh():
    return plsc.VectorSubcoreMesh(core_axis_name="c", subcore_axis_name="s")


_SC_PARAMS = pltpu.CompilerParams(use_tc_tiling_on_sc=False)


def _gather(src, dst, tsrc, tdst):
    @functools.partial(
        pl.kernel,
        out_type=(jax.ShapeDtypeStruct((_E, _H), _f32),
                  jax.ShapeDtypeStruct((_E, 2 * _H), _f32)),
        mesh=_sc_mesh(),
        scratch_types=[
            pltpu.VMEM((_C,), jnp.int32),
            pltpu.VMEM((_C,), jnp.int32),
            pltpu.VMEM((_C, _H), _f32),
            pltpu.VMEM((_C, 2 * _H), _f32),
            pltpu.SemaphoreType.DMA,
            pltpu.SemaphoreType.DMA,
        ],
        compiler_params=_SC_PARAMS,
    )
    def gather_k(src_hbm, dst_hbm, ts_hbm, td_hbm, gs_hbm, gd_hbm,
                 idx_s, idx_d, row_s, row_d, sem_a, sem_b):
        wid = lax.axis_index("s") * _NC + lax.axis_index("c")
        base = wid * _PER_W

        @pl.loop(0, _NCHUNK)
        def _(j):
            off = base + j * _C
            pltpu.sync_copy(src_hbm.at[pl.ds(off, _C)], idx_s)
            pltpu.sync_copy(dst_hbm.at[pl.ds(off, _C)], idx_d)
            cp_a = pltpu.async_copy(ts_hbm.at[idx_s], row_s, sem_a)
            cp_b = pltpu.async_copy(td_hbm.at[idx_d], row_d, sem_b)
            cp_a.wait()
            cp_b.wait()
            pltpu.sync_copy(row_s, gs_hbm.at[pl.ds(off, _C)])
            pltpu.sync_copy(row_d, gd_hbm.at[pl.ds(off, _C)])

    return gather_k(src, dst, tsrc, tdst)


def _edge_body(gs_ref, gd_ref, fe_ref, eW1a_ref, eW2_ref, eb2_ref,
               n1W1e_ref, n1W2_ref, n1b2_ref, en_ref, mext_ref):
    gd = gd_ref[...]
    h = jnp.maximum(gs_ref[...] + gd[:, :_H] + _dot(fe_ref[...], eW1a_ref[...]),
                    0.0)
    en = _dot(h, eW2_ref[...]) + eb2_ref[...]
    mh = jnp.maximum(gd[:, _H:] + _dot(en, n1W1e_ref[...]), 0.0)
    m = _dot(mh, n1W2_ref[...]) + n1b2_ref[...]
    en_ref[...] = en
    mext_ref[...] = jnp.concatenate(
        [m, jnp.ones((_BE, 1), _f32), jnp.zeros((_BE, 15), _f32)], axis=1)


def _edge(gsrc, gdst, fe, eW1a, eW2, eb2, n1W1e, n1W2, n1b2):
    full = lambda shp: pl.BlockSpec(shp, lambda i: (0,) * len(shp))
    return pl.pallas_call(
        _edge_body,
        grid=(_E // _BE,),
        in_specs=[
            pl.BlockSpec((_BE, _H), lambda i: (i, 0)),
            pl.BlockSpec((_BE, 2 * _H), lambda i: (i, 0)),
            pl.BlockSpec((_BE, _DE), lambda i: (i, 0)),
            full((_DE, _H)), full((_H, _DE)), full((1, _DE)),
            full((_DE, _H)), full((_H, _H)), full((1, _H)),
        ],
        out_specs=[
            pl.BlockSpec((_BE, _DE), lambda i: (i, 0)),
            pl.BlockSpec((_BE, 48), lambda i: (i, 0)),
        ],
        out_shape=[
            jax.ShapeDtypeStruct((_E, _DE), _f32),
            jax.ShapeDtypeStruct((_E, 48), _f32),
        ],
    )(gsrc, gdst, fe, eW1a, eW2, eb2, n1W1e, n1W2, n1b2)


def _scatter(src, mext, zeros):
    rows_per_sub = _N // _NS

    @functools.partial(
        pl.kernel,
        out_type=jax.ShapeDtypeStruct((_NC, _N, 48), _f32),
        mesh=_sc_mesh(),
        scratch_types=[
            pltpu.VMEM((_C,), jnp.int32),
            pltpu.VMEM((_C, 48), _f32),
            pltpu.VMEM_SHARED((_N, 48), _f32),
        ],
        compiler_params=_SC_PARAMS,
    )
    def scatter_k(src_hbm, m_hbm, z_hbm, out_hbm, idx_v, row_v, acc):
        cid = lax.axis_index("c")
        sid = lax.axis_index("s")
        r0 = sid * rows_per_sub
        pltpu.sync_copy(z_hbm.at[pl.ds(r0, rows_per_sub)],
                        acc.at[pl.ds(r0, rows_per_sub)])
        plsc.subcore_barrier()
        wid = sid * _NC + cid
        base = wid * _PER_W

        @pl.loop(0, _NCHUNK)
        def _(j):
            off = base + j * _C
            pltpu.sync_copy(src_hbm.at[pl.ds(off, _C)], idx_v)
            pltpu.sync_copy(m_hbm.at[pl.ds(off, _C)], row_v)
            pltpu.sync_copy(row_v, acc.at[idx_v], add=True)

        plsc.subcore_barrier()
        pltpu.sync_copy(acc.at[pl.ds(r0, rows_per_sub)],
                        out_hbm.at[cid, pl.ds(r0, rows_per_sub)])

    return scatter_k(src, mext, zeros)


def _node_body(p_ref, xp2_ref, b_ref, n2W1a_ref, n2W2_ref, n2b2_ref,
               xn_ref, gs_ref, gc_ref):
    p = p_ref[...]
    aggs = p[0, :, :_H] + p[1, :, :_H]
    cnt = p[0, :, _H:_H + 1] + p[1, :, _H:_H + 1]
    agg = aggs / jnp.maximum(cnt, 1.0)
    xh = jnp.maximum(xp2_ref[...] + _dot(agg, n2W1a_ref[...]), 0.0)
    xn = _dot(xh, n2W2_ref[...]) + n2b2_ref[...]
    xn_ref[...] = xn
    iota = lax.broadcasted_iota(jnp.int32, (_BN, _B), 1)
    oh = (b_ref[...] == iota).astype(_f32)
    gs_blk = _dot_t(oh, xn)
    gc_blk = _dot_t(oh, jnp.ones((_BN, _DF), _f32))

    @pl.when(pl.program_id(0) == 0)
    def _():
        gs_ref[...] = gs_blk
        gc_ref[...] = gc_blk

    @pl.when(pl.program_id(0) != 0)
    def _():
        gs_ref[...] += gs_blk
        gc_ref[...] += gc_blk


def _node(p, xp2, bids2, n2W1a, n2W2, n2b2):
    full = lambda shp: pl.BlockSpec(shp, lambda i: (0,) * len(shp))
    return pl.pallas_call(
        _node_body,
        grid=(_N // _BN,),
        in_specs=[
            pl.BlockSpec((_NC, _BN, 48), lambda i: (0, i, 0)),
            pl.BlockSpec((_BN, _H), lambda i: (i, 0)),
            pl.BlockSpec((_BN, 1), lambda i: (i, 0)),
            full((_H, _H)), full((_H, _DF)), full((1, _DF)),
        ],
        out_specs=[
            pl.BlockSpec((_BN, _DF), lambda i: (i, 0)),
            full((_B, _DF)),
            full((_B, _DF)),
        ],
        out_shape=[
            jax.ShapeDtypeStruct((_N, _DF), _f32),
            jax.ShapeDtypeStruct((_B, _DF), _f32),
            jax.ShapeDtypeStruct((_B, _DF), _f32),
        ],
    )(p, xp2, bids2, n2W1a, n2W2, n2b2)


def _glob_body(u_ref, gs_ref, gc_ref, gW1u_ref, gW1m_ref, gb1_ref,
               gW2_ref, gb2_ref, un_ref):
    mean = gs_ref[...] / jnp.maximum(gc_ref[...], 1.0)
    h = jnp.maximum(_dot(u_ref[...], gW1u_ref[...]) +
                    _dot(mean, gW1m_ref[...]) + gb1_ref[...], 0.0)
    un_ref[...] = _dot(h, gW2_ref[...]) + gb2_ref[...]


def _glob(u, gs, gc, gW1u, gW1m, gb1, gW2, gb2):
    return pl.pallas_call(
        _glob_body,
        out_shape=jax.ShapeDtypeStruct((_B, _DU), _f32),
    )(u, gs, gc, gW1u, gW1m, gb1, gW2, gb2)


def kernel(features_of_nodes, node_type_ids, node_ids_for_edges,
           features_of_edges, edge_type_ids, global_features, batch_ids,
           eW1, eb1, eW2, eb2,
           n1W1, n1b1, n1W2, n1b2,
           n2W1, n2b1, n2W2, n2b2,
           gW1, gb1, gW2, gb2):
    x = features_of_nodes
    u = global_features
    src = node_ids_for_edges[0].astype(jnp.int32)
    dst = node_ids_for_edges[1].astype(jnp.int32)
    bids2 = batch_ids.astype(jnp.int32).reshape(_N, 1)

    r1 = lambda v: v.reshape(1, -1)
    tsrc, tdst, xp2 = _precompute(
        x, bids2, u,
        eW1[:_DF], eW1[_DF:2 * _DF], eW1[2 * _DF + _DE:],
        n1W1[:_DF], n2W1[:_DF], n2W1[_DF + _H:],
        r1(eb1), r1(n1b1), r1(n2b1))

    gsrc, gdst = _gather(src, dst, tsrc, tdst)

    e_new, mext = _edge(gsrc, gdst, features_of_edges,
                        eW1[2 * _DF:2 * _DF + _DE], eW2, r1(eb2),
                        n1W1[_DF:], n1W2, r1(n1b2))

    p = _scatter(src, mext, jnp.zeros((_N, 48), _f32))

    x_new, gs, gc = _node(p, xp2, bids2, n2W1[_DF:_DF + _H], n2W2, r1(n2b2))

    u_new = _glob(u, gs, gc, gW1[:_DU], gW1[_DU:], r1(gb1), gW2, r1(gb2))

    return (x_new, e_new, u_new)


# speedup vs baseline: 3.2707x; 1.0947x over previous
import functools

import jax
import jax.numpy as jnp
from jax import lax
from jax.experimental import pallas as pl
from jax.experimental.pallas import tpu as pltpu
from jax.experimental.pallas import tpu_sc as plsc

_N = 10000
_E = 320000
_B = 16
_DF = 128
_DE = 16
_DU = 32
_H = 32

_NC = 2
_NS = 16
_NW = _NC * _NS
_C = 80
_PER_W = _E // _NW
_NCHUNK = _PER_W // _C

_BN = 2000
_BE = 2000

_f32 = jnp.float32


def _dot(a, b):
    return lax.dot_general(a, b, (((1,), (0,)), ((), ())),
                           preferred_element_type=_f32,
                           precision=lax.Precision.HIGHEST)


def _dot_t(a, b):
    return lax.dot_general(a, b, (((0,), (0,)), ((), ())),
                           preferred_element_type=_f32,
                           precision=lax.Precision.HIGHEST)


def _pre_body(x_ref, b_ref, u_ref, eW1xs_ref, eW1xd_ref, eW1u_ref, n1W1x_ref,
              n2W1x_ref, n2W1u_ref, eb1_ref, n1b1_ref, n2b1_ref,
              tsrc_ref, xp2_ref):
    xb = x_ref[...]
    bid = b_ref[...]
    iota = lax.broadcasted_iota(jnp.int32, (_BN, _B), 1)
    oh = (bid == iota).astype(_f32)
    ue = _dot(u_ref[...], eW1u_ref[...])
    un = _dot(u_ref[...], n2W1u_ref[...])
    tsrc = _dot(xb, eW1xs_ref[...]) + _dot(oh, ue) + eb1_ref[...]
    tsrc_ref[...] = jnp.concatenate(
        [tsrc,
         _dot(xb, eW1xd_ref[...]),
         _dot(xb, n1W1x_ref[...]) + n1b1_ref[...],
         jnp.zeros((_BN, _H), _f32)], axis=1)
    xp2_ref[...] = _dot(xb, n2W1x_ref[...]) + _dot(oh, un) + n2b1_ref[...]


def _precompute(x, bids2, u, eW1xs, eW1xd, eW1u, n1W1x, n2W1x, n2W1u,
                eb1, n1b1, n2b1):
    full = lambda shp: pl.BlockSpec(shp, lambda i: (0,) * len(shp))
    return pl.pallas_call(
        _pre_body,
        grid=(_N // _BN,),
        in_specs=[
            pl.BlockSpec((_BN, _DF), lambda i: (i, 0)),
            pl.BlockSpec((_BN, 1), lambda i: (i, 0)),
            full((_B, _DU)),
            full((_DF, _H)), full((_DF, _H)), full((_DU, _H)),
            full((_DF, _H)), full((_DF, _H)), full((_DU, _H)),
            full((1, _H)), full((1, _H)), full((1, _H)),
        ],
        out_specs=[
            pl.BlockSpec((_BN, _DF), lambda i: (i, 0)),
            pl.BlockSpec((_BN, _H), lambda i: (i, 0)),
        ],
        out_shape=[
            jax.ShapeDtypeStruct((_N, _DF), _f32),
            jax.ShapeDtypeStruct((_N, _H), _f32),
        ],
    )(x, bids2, u, eW1xs, eW1xd, eW1u, n1W1x, n2W1x, n2W1u, eb1, n1b1, n2b1)


def _sc_mesh():
    return plsc.VectorSubcoreMesh(core_axis_name="c", subcore_axis_name="s")


def _gather(src, dst, table):
    @functools.partial(
        pl.kernel,
        out_type=(jax.ShapeDtypeStruct((_E, _DF), _f32),
                  jax.ShapeDtypeStruct((_E, _DF), _f32)),
        mesh=_sc_mesh(),
        scratch_types=[
            pltpu.VMEM((_C,), jnp.int32),
            pltpu.VMEM((_C,), jnp.int32),
            pltpu.VMEM((_C, _DF), _f32),
            pltpu.VMEM((_C, _DF), _f32),
            pltpu.SemaphoreType.DMA,
            pltpu.SemaphoreType.DMA,
        ],
    )
    def gather_k(src_hbm, dst_hbm, t_hbm, gs_hbm, gd_hbm,
                 idx_s, idx_d, row_s, row_d, sem_a, sem_b):
        wid = lax.axis_index("s") * _NC + lax.axis_index("c")
        base = wid * _PER_W

        @pl.loop(0, _NCHUNK)
        def _(j):
            off = base + j * _C
            pltpu.sync_copy(src_hbm.at[pl.ds(off, _C)], idx_s)
            pltpu.sync_copy(dst_hbm.at[pl.ds(off, _C)], idx_d)
            cp_a = pltpu.async_copy(t_hbm.at[idx_s], row_s, sem_a)
            cp_b = pltpu.async_copy(t_hbm.at[idx_d], row_d, sem_b)
            cp_a.wait()
            cp_b.wait()
            pltpu.sync_copy(row_s, gs_hbm.at[pl.ds(off, _C)])
            pltpu.sync_copy(row_d, gd_hbm.at[pl.ds(off, _C)])

    return gather_k(src, dst, table)


def _edge_body(gs_ref, gd_ref, fe_ref, eW1a_ref, eW2_ref, eb2_ref,
               n1W1e_ref, n1W2_ref, n1b2_ref, en_ref, mext_ref):
    gd = gd_ref[...]
    h = jnp.maximum(gs_ref[:, :_H] + gd[:, _H:2 * _H]
                    + _dot(fe_ref[...], eW1a_ref[...]), 0.0)
    en = _dot(h, eW2_ref[...]) + eb2_ref[...]
    mh = jnp.maximum(gd[:, 2 * _H:3 * _H] + _dot(en, n1W1e_ref[...]), 0.0)
    m = _dot(mh, n1W2_ref[...]) + n1b2_ref[...]
    en_ref[...] = en
    mext_ref[...] = jnp.concatenate(
        [m, jnp.ones((_BE, 1), _f32), jnp.zeros((_BE, 95), _f32)], axis=1)


def _edge(gsrc, gdst, fe, eW1a, eW2, eb2, n1W1e, n1W2, n1b2):
    full = lambda shp: pl.BlockSpec(shp, lambda i: (0,) * len(shp))
    return pl.pallas_call(
        _edge_body,
        grid=(_E // _BE,),
        in_specs=[
            pl.BlockSpec((_BE, _DF), lambda i: (i, 0)),
            pl.BlockSpec((_BE, _DF), lambda i: (i, 0)),
            pl.BlockSpec((_BE, _DE), lambda i: (i, 0)),
            full((_DE, _H)), full((_H, _DE)), full((1, _DE)),
            full((_DE, _H)), full((_H, _H)), full((1, _H)),
        ],
        out_specs=[
            pl.BlockSpec((_BE, _DE), lambda i: (i, 0)),
            pl.BlockSpec((_BE, _DF), lambda i: (i, 0)),
        ],
        out_shape=[
            jax.ShapeDtypeStruct((_E, _DE), _f32),
            jax.ShapeDtypeStruct((_E, _DF), _f32),
        ],
    )(gsrc, gdst, fe, eW1a, eW2, eb2, n1W1e, n1W2, n1b2)


_NP = 10240


def _scatter(src, mext, zeros):
    rows_per_sub = _NP // _NS

    @functools.partial(
        pl.kernel,
        out_type=jax.ShapeDtypeStruct((_NC, _NP, _DF), _f32),
        mesh=_sc_mesh(),
        scratch_types=[
            pltpu.VMEM((_C,), jnp.int32),
            pltpu.VMEM((_C, _DF), _f32),
            pltpu.VMEM_SHARED((_NP, _DF), _f32),
        ],
    )
    def scatter_k(src_hbm, m_hbm, z_hbm, out_hbm, idx_v, row_v, acc):
        cid = lax.axis_index("c")
        sid = lax.axis_index("s")
        r0 = sid * rows_per_sub
        pltpu.sync_copy(z_hbm.at[pl.ds(r0, rows_per_sub)],
                        acc.at[pl.ds(r0, rows_per_sub)])
        plsc.subcore_barrier()
        wid = sid * _NC + cid
        base = wid * _PER_W

        @pl.loop(0, _NCHUNK)
        def _(j):
            off = base + j * _C
            pltpu.sync_copy(src_hbm.at[pl.ds(off, _C)], idx_v)
            pltpu.sync_copy(m_hbm.at[pl.ds(off, _C)], row_v)
            pltpu.sync_copy(row_v, acc.at[idx_v], add=True)

        plsc.subcore_barrier()
        pltpu.sync_copy(acc.at[pl.ds(r0, rows_per_sub)],
                        out_hbm.at[cid, pl.ds(r0, rows_per_sub)])

    return scatter_k(src, mext, zeros)


def _node_body(p_ref, xp2_ref, b_ref, n2W1a_ref, n2W2_ref, n2b2_ref,
               xn_ref, gs_ref, gc_ref):
    p = p_ref[...]
    aggs = p[0, :, :_H] + p[1, :, :_H]
    cnt = p[0, :, _H:_H + 1] + p[1, :, _H:_H + 1]
    agg = aggs / jnp.maximum(cnt, 1.0)
    xh = jnp.maximum(xp2_ref[...] + _dot(agg, n2W1a_ref[...]), 0.0)
    xn = _dot(xh, n2W2_ref[...]) + n2b2_ref[...]
    xn_ref[...] = xn
    iota = lax.broadcasted_iota(jnp.int32, (_BN, _B), 1)
    oh = (b_ref[...] == iota).astype(_f32)
    gs_blk = _dot_t(oh, xn)
    gc_blk = _dot_t(oh, jnp.ones((_BN, _DF), _f32))

    @pl.when(pl.program_id(0) == 0)
    def _():
        gs_ref[...] = gs_blk
        gc_ref[...] = gc_blk

    @pl.when(pl.program_id(0) != 0)
    def _():
        gs_ref[...] += gs_blk
        gc_ref[...] += gc_blk


def _node(p, xp2, bids2, n2W1a, n2W2, n2b2):
    full = lambda shp: pl.BlockSpec(shp, lambda i: (0,) * len(shp))
    return pl.pallas_call(
        _node_body,
        grid=(_N // _BN,),
        in_specs=[
            pl.BlockSpec((_NC, _BN, _DF), lambda i: (0, i, 0)),
            pl.BlockSpec((_BN, _H), lambda i: (i, 0)),
            pl.BlockSpec((_BN, 1), lambda i: (i, 0)),
            full((_H, _H)), full((_H, _DF)), full((1, _DF)),
        ],
        out_specs=[
            pl.BlockSpec((_BN, _DF), lambda i: (i, 0)),
            full((_B, _DF)),
            full((_B, _DF)),
        ],
        out_shape=[
            jax.ShapeDtypeStruct((_N, _DF), _f32),
            jax.ShapeDtypeStruct((_B, _DF), _f32),
            jax.ShapeDtypeStruct((_B, _DF), _f32),
        ],
    )(p, xp2, bids2, n2W1a, n2W2, n2b2)


def _glob_body(u_ref, gs_ref, gc_ref, gW1u_ref, gW1m_ref, gb1_ref,
               gW2_ref, gb2_ref, un_ref):
    mean = gs_ref[...] / jnp.maximum(gc_ref[...], 1.0)
    h = jnp.maximum(_dot(u_ref[...], gW1u_ref[...]) +
                    _dot(mean, gW1m_ref[...]) + gb1_ref[...], 0.0)
    un_ref[...] = _dot(h, gW2_ref[...]) + gb2_ref[...]


def _glob(u, gs, gc, gW1u, gW1m, gb1, gW2, gb2):
    return pl.pallas_call(
        _glob_body,
        out_shape=jax.ShapeDtypeStruct((_B, _DU), _f32),
    )(u, gs, gc, gW1u, gW1m, gb1, gW2, gb2)


def kernel(features_of_nodes, node_type_ids, node_ids_for_edges,
           features_of_edges, edge_type_ids, global_features, batch_ids,
           eW1, eb1, eW2, eb2,
           n1W1, n1b1, n1W2, n1b2,
           n2W1, n2b1, n2W2, n2b2,
           gW1, gb1, gW2, gb2):
    x = features_of_nodes
    u = global_features
    src = node_ids_for_edges[0].astype(jnp.int32)
    dst = node_ids_for_edges[1].astype(jnp.int32)
    bids2 = batch_ids.astype(jnp.int32).reshape(_N, 1)

    r1 = lambda v: v.reshape(1, -1)
    table, xp2 = _precompute(
        x, bids2, u,
        eW1[:_DF], eW1[_DF:2 * _DF], eW1[2 * _DF + _DE:],
        n1W1[:_DF], n2W1[:_DF], n2W1[_DF + _H:],
        r1(eb1), r1(n1b1), r1(n2b1))

    gsrc, gdst = _gather(src, dst, table)

    e_new, mext = _edge(gsrc, gdst, features_of_edges,
                        eW1[2 * _DF:2 * _DF + _DE], eW2, r1(eb2),
                        n1W1[_DF:], n1W2, r1(n1b2))

    p = _scatter(src, mext, jnp.zeros((_NP, _DF), _f32))

    x_new, gs, gc = _node(p, xp2, bids2, n2W1[_DF:_DF + _H], n2W2, r1(n2b2))

    u_new = _glob(u, gs, gc, gW1[:_DU], gW1[_DU:], r1(gb1), gW2, r1(gb2))

    return (x_new, e_new, u_new)


# speedup vs baseline: 5.5483x; 1.6964x over previous
import functools

import jax
import jax.numpy as jnp
from jax import lax
from jax.experimental import pallas as pl
from jax.experimental.pallas import tpu as pltpu
from jax.experimental.pallas import tpu_sc as plsc

_N = 10000
_E = 320000
_B = 16
_DF = 128
_DE = 16
_DU = 32
_H = 32

_NC = 2
_NS = 16
_NW = _NC * _NS
_C = 80
_PER_W = _E // _NW
_NCHUNK = _PER_W // _C

_BN = 2000
_BE = 4000

_f32 = jnp.float32


def _dot(a, b, precision=lax.Precision.HIGHEST):
    return lax.dot_general(a, b, (((1,), (0,)), ((), ())),
                           preferred_element_type=_f32,
                           precision=precision)


def _dot_h(a, b):
    return _dot(a, b, precision=lax.Precision.DEFAULT)


def _dot_t(a, b):
    return lax.dot_general(a, b, (((0,), (0,)), ((), ())),
                           preferred_element_type=_f32,
                           precision=lax.Precision.HIGHEST)


def _pre_body(x_ref, b_ref, u_ref, eW1xs_ref, eW1xd_ref, eW1u_ref, n1W1x_ref,
              n2W1x_ref, n2W1u_ref, eb1_ref, n1b1_ref, n2b1_ref,
              tsrc_ref, xp2_ref):
    xb = x_ref[...]
    bid = b_ref[...]
    iota = lax.broadcasted_iota(jnp.int32, (_BN, _B), 1)
    oh = (bid == iota).astype(_f32)
    ue = _dot(u_ref[...], eW1u_ref[...])
    un = _dot(u_ref[...], n2W1u_ref[...])
    tsrc = _dot(xb, eW1xs_ref[...]) + _dot(oh, ue) + eb1_ref[...]
    tsrc_ref[...] = jnp.concatenate(
        [tsrc,
         _dot(xb, eW1xd_ref[...]),
         _dot(xb, n1W1x_ref[...]) + n1b1_ref[...],
         jnp.zeros((_BN, _H), _f32)], axis=1)
    xp2_ref[...] = _dot(xb, n2W1x_ref[...]) + _dot(oh, un) + n2b1_ref[...]


def _precompute(x, bids2, u, eW1xs, eW1xd, eW1u, n1W1x, n2W1x, n2W1u,
                eb1, n1b1, n2b1):
    full = lambda shp: pl.BlockSpec(shp, lambda i: (0,) * len(shp))
    return pl.pallas_call(
        _pre_body,
        grid=(_N // _BN,),
        in_specs=[
            pl.BlockSpec((_BN, _DF), lambda i: (i, 0)),
            pl.BlockSpec((_BN, 1), lambda i: (i, 0)),
            full((_B, _DU)),
            full((_DF, _H)), full((_DF, _H)), full((_DU, _H)),
            full((_DF, _H)), full((_DF, _H)), full((_DU, _H)),
            full((1, _H)), full((1, _H)), full((1, _H)),
        ],
        out_specs=[
            pl.BlockSpec((_BN, _DF), lambda i: (i, 0)),
            pl.BlockSpec((_BN, _H), lambda i: (i, 0)),
        ],
        out_shape=[
            jax.ShapeDtypeStruct((_N, _DF), _f32),
            jax.ShapeDtypeStruct((_N, _H), _f32),
        ],
    )(x, bids2, u, eW1xs, eW1xd, eW1u, n1W1x, n2W1x, n2W1u, eb1, n1b1, n2b1)


def _sc_mesh():
    return plsc.VectorSubcoreMesh(core_axis_name="c", subcore_axis_name="s")


def _gather(src, dst, table):
    @functools.partial(
        pl.kernel,
        out_type=(jax.ShapeDtypeStruct((_E, _DF), _f32),
                  jax.ShapeDtypeStruct((_E, _DF), _f32)),
        mesh=_sc_mesh(),
        scratch_types=[
            pltpu.VMEM((_C,), jnp.int32),
            pltpu.VMEM((_C,), jnp.int32),
            pltpu.VMEM((_C, _DF), _f32),
            pltpu.VMEM((_C, _DF), _f32),
            pltpu.SemaphoreType.DMA,
            pltpu.SemaphoreType.DMA,
        ],
    )
    def gather_k(src_hbm, dst_hbm, t_hbm, gs_hbm, gd_hbm,
                 idx_s, idx_d, row_s, row_d, sem_a, sem_b):
        wid = lax.axis_index("s") * _NC + lax.axis_index("c")
        base = wid * _PER_W

        @pl.loop(0, _NCHUNK)
        def _(j):
            off = base + j * _C
            pltpu.sync_copy(src_hbm.at[pl.ds(off, _C)], idx_s)
            pltpu.sync_copy(dst_hbm.at[pl.ds(off, _C)], idx_d)
            cp_a = pltpu.async_copy(t_hbm.at[idx_s], row_s, sem_a)
            cp_b = pltpu.async_copy(t_hbm.at[idx_d], row_d, sem_b)
            cp_a.wait()
            cp_b.wait()
            pltpu.sync_copy(row_s, gs_hbm.at[pl.ds(off, _C)])
            pltpu.sync_copy(row_d, gd_hbm.at[pl.ds(off, _C)])

    return gather_k(src, dst, table)


def _edge_body(gs_ref, gd_ref, fe_ref, eW1a_ref, eW2_ref, eb2_ref,
               n1W1e_ref, n1W2_ref, n1b2_ref, en_ref, mext_ref):
    gd = gd_ref[...]
    h = jnp.maximum(gs_ref[:, :_H] + gd[:, _H:2 * _H]
                    + _dot_h(fe_ref[...], eW1a_ref[...]), 0.0)
    en = _dot_h(h, eW2_ref[...]) + eb2_ref[...]
    mh = jnp.maximum(gd[:, 2 * _H:3 * _H] + _dot_h(en, n1W1e_ref[...]), 0.0)
    m = _dot_h(mh, n1W2_ref[...]) + n1b2_ref[...]
    en_ref[...] = en
    mext_ref[...] = jnp.concatenate(
        [m, jnp.ones((_BE, 1), _f32), jnp.zeros((_BE, 95), _f32)], axis=1)


def _edge(gsrc, gdst, fe, eW1a, eW2, eb2, n1W1e, n1W2, n1b2):
    full = lambda shp: pl.BlockSpec(shp, lambda i: (0,) * len(shp))
    return pl.pallas_call(
        _edge_body,
        grid=(_E // _BE,),
        in_specs=[
            pl.BlockSpec((_BE, _DF), lambda i: (i, 0)),
            pl.BlockSpec((_BE, _DF), lambda i: (i, 0)),
            pl.BlockSpec((_BE, _DE), lambda i: (i, 0)),
            full((_DE, _H)), full((_H, _DE)), full((1, _DE)),
            full((_DE, _H)), full((_H, _H)), full((1, _H)),
        ],
        out_specs=[
            pl.BlockSpec((_BE, _DE), lambda i: (i, 0)),
            pl.BlockSpec((_BE, _DF), lambda i: (i, 0)),
        ],
        out_shape=[
            jax.ShapeDtypeStruct((_E, _DE), _f32),
            jax.ShapeDtypeStruct((_E, _DF), _f32),
        ],
    )(gsrc, gdst, fe, eW1a, eW2, eb2, n1W1e, n1W2, n1b2)


_NP = 10240


def _scatter(src, mext, zeros):
    rows_per_sub = _NP // _NS

    @functools.partial(
        pl.kernel,
        out_type=jax.ShapeDtypeStruct((_NC, _NP, _DF), _f32),
        mesh=_sc_mesh(),
        scratch_types=[
            pltpu.VMEM((_C,), jnp.int32),
            pltpu.VMEM((_C, _DF), _f32),
            pltpu.VMEM_SHARED((_NP, _DF), _f32),
        ],
    )
    def scatter_k(src_hbm, m_hbm, z_hbm, out_hbm, idx_v, row_v, acc):
        cid = lax.axis_index("c")
        sid = lax.axis_index("s")
        r0 = sid * rows_per_sub
        pltpu.sync_copy(z_hbm.at[pl.ds(r0, rows_per_sub)],
                        acc.at[pl.ds(r0, rows_per_sub)])
        plsc.subcore_barrier()
        wid = sid * _NC + cid
        base = wid * _PER_W

        @pl.loop(0, _NCHUNK)
        def _(j):
            off = base + j * _C
            pltpu.sync_copy(src_hbm.at[pl.ds(off, _C)], idx_v)
            pltpu.sync_copy(m_hbm.at[pl.ds(off, _C)], row_v)
            pltpu.sync_copy(row_v, acc.at[idx_v], add=True)

        plsc.subcore_barrier()
        pltpu.sync_copy(acc.at[pl.ds(r0, rows_per_sub)],
                        out_hbm.at[cid, pl.ds(r0, rows_per_sub)])

    return scatter_k(src, mext, zeros)


def _node_body(p_ref, xp2_ref, b_ref, n2W1a_ref, n2W2_ref, n2b2_ref,
               xn_ref, gs_ref, gc_ref):
    p = p_ref[...]
    aggs = p[0, :, :_H] + p[1, :, :_H]
    cnt = p[0, :, _H:_H + 1] + p[1, :, _H:_H + 1]
    agg = aggs / jnp.maximum(cnt, 1.0)
    xh = jnp.maximum(xp2_ref[...] + _dot(agg, n2W1a_ref[...]), 0.0)
    xn = _dot(xh, n2W2_ref[...]) + n2b2_ref[...]
    xn_ref[...] = xn
    iota = lax.broadcasted_iota(jnp.int32, (_BN, _B), 1)
    oh = (b_ref[...] == iota).astype(_f32)
    gs_blk = _dot_t(oh, xn)
    gc_blk = _dot_t(oh, jnp.ones((_BN, _DF), _f32))

    @pl.when(pl.program_id(0) == 0)
    def _():
        gs_ref[...] = gs_blk
        gc_ref[...] = gc_blk

    @pl.when(pl.program_id(0) != 0)
    def _():
        gs_ref[...] += gs_blk
        gc_ref[...] += gc_blk


def _node(p, xp2, bids2, n2W1a, n2W2, n2b2):
    full = lambda shp: pl.BlockSpec(shp, lambda i: (0,) * len(shp))
    return pl.pallas_call(
        _node_body,
        grid=(_N // _BN,),
        in_specs=[
            pl.BlockSpec((_NC, _BN, _DF), lambda i: (0, i, 0)),
            pl.BlockSpec((_BN, _H), lambda i: (i, 0)),
            pl.BlockSpec((_BN, 1), lambda i: (i, 0)),
            full((_H, _H)), full((_H, _DF)), full((1, _DF)),
        ],
        out_specs=[
            pl.BlockSpec((_BN, _DF), lambda i: (i, 0)),
            full((_B, _DF)),
            full((_B, _DF)),
        ],
        out_shape=[
            jax.ShapeDtypeStruct((_N, _DF), _f32),
            jax.ShapeDtypeStruct((_B, _DF), _f32),
            jax.ShapeDtypeStruct((_B, _DF), _f32),
        ],
    )(p, xp2, bids2, n2W1a, n2W2, n2b2)


def _glob_body(u_ref, gs_ref, gc_ref, gW1u_ref, gW1m_ref, gb1_ref,
               gW2_ref, gb2_ref, un_ref):
    mean = gs_ref[...] / jnp.maximum(gc_ref[...], 1.0)
    h = jnp.maximum(_dot(u_ref[...], gW1u_ref[...]) +
                    _dot(mean, gW1m_ref[...]) + gb1_ref[...], 0.0)
    un_ref[...] = _dot(h, gW2_ref[...]) + gb2_ref[...]


def _glob(u, gs, gc, gW1u, gW1m, gb1, gW2, gb2):
    return pl.pallas_call(
        _glob_body,
        out_shape=jax.ShapeDtypeStruct((_B, _DU), _f32),
    )(u, gs, gc, gW1u, gW1m, gb1, gW2, gb2)


def kernel(features_of_nodes, node_type_ids, node_ids_for_edges,
           features_of_edges, edge_type_ids, global_features, batch_ids,
           eW1, eb1, eW2, eb2,
           n1W1, n1b1, n1W2, n1b2,
           n2W1, n2b1, n2W2, n2b2,
           gW1, gb1, gW2, gb2):
    x = features_of_nodes
    u = global_features
    src = node_ids_for_edges[0].astype(jnp.int32)
    dst = node_ids_for_edges[1].astype(jnp.int32)
    bids2 = batch_ids.astype(jnp.int32).reshape(_N, 1)

    r1 = lambda v: v.reshape(1, -1)
    table, xp2 = _precompute(
        x, bids2, u,
        eW1[:_DF], eW1[_DF:2 * _DF], eW1[2 * _DF + _DE:],
        n1W1[:_DF], n2W1[:_DF], n2W1[_DF + _H:],
        r1(eb1), r1(n1b1), r1(n2b1))

    gsrc, gdst = _gather(src, dst, table)

    e_new, mext = _edge(gsrc, gdst, features_of_edges,
                        eW1[2 * _DF:2 * _DF + _DE], eW2, r1(eb2),
                        n1W1[_DF:], n1W2, r1(n1b2))

    p = _scatter(src, mext, jnp.zeros((_NP, _DF), _f32))

    x_new, gs, gc = _node(p, xp2, bids2, n2W1[_DF:_DF + _H], n2W2, r1(n2b2))

    u_new = _glob(u, gs, gc, gW1[:_DU], gW1[_DU:], r1(gb1), gW2, r1(gb2))

    return (x_new, e_new, u_new)


# speedup vs baseline: 5.7347x; 1.0336x over previous
import functools

import jax
import jax.numpy as jnp
from jax import lax
from jax.experimental import pallas as pl
from jax.experimental.pallas import tpu as pltpu
from jax.experimental.pallas import tpu_sc as plsc

_N = 10000
_E = 320000
_B = 16
_DF = 128
_DE = 16
_DU = 32
_H = 32

_NC = 2
_NS = 16
_NW = _NC * _NS
_C = 80
_PER_W = _E // _NW
_NCHUNK = _PER_W // _C

_BN = 2000
_BE = 4000

_f32 = jnp.float32


def _dot(a, b, precision=lax.Precision.HIGHEST):
    return lax.dot_general(a, b, (((1,), (0,)), ((), ())),
                           preferred_element_type=_f32,
                           precision=precision)


def _dot_h(a, b):
    return _dot(a, b, precision=lax.Precision.DEFAULT)


def _dot_t(a, b):
    return lax.dot_general(a, b, (((0,), (0,)), ((), ())),
                           preferred_element_type=_f32,
                           precision=lax.Precision.HIGHEST)


def _pre_body(x_ref, b_ref, u_ref, eW1xs_ref, eW1xd_ref, eW1u_ref, n1W1x_ref,
              n2W1x_ref, n2W1u_ref, eb1_ref, n1b1_ref, n2b1_ref,
              tsrc_ref, xp2_ref):
    xb = x_ref[...]
    bid = b_ref[...]
    iota = lax.broadcasted_iota(jnp.int32, (_BN, _B), 1)
    oh = (bid == iota).astype(_f32)
    ue = _dot(u_ref[...], eW1u_ref[...])
    un = _dot(u_ref[...], n2W1u_ref[...])
    tsrc = _dot(xb, eW1xs_ref[...]) + _dot(oh, ue) + eb1_ref[...]
    tsrc_ref[...] = jnp.concatenate(
        [tsrc,
         _dot(xb, eW1xd_ref[...]),
         _dot(xb, n1W1x_ref[...]) + n1b1_ref[...],
         jnp.zeros((_BN, _H), _f32)], axis=1)
    xp2_ref[...] = _dot(xb, n2W1x_ref[...]) + _dot(oh, un) + n2b1_ref[...]


def _precompute(x, bids2, u, eW1xs, eW1xd, eW1u, n1W1x, n2W1x, n2W1u,
                eb1, n1b1, n2b1):
    full = lambda shp: pl.BlockSpec(shp, lambda i: (0,) * len(shp))
    return pl.pallas_call(
        _pre_body,
        grid=(_N // _BN,),
        in_specs=[
            pl.BlockSpec((_BN, _DF), lambda i: (i, 0)),
            pl.BlockSpec((_BN, 1), lambda i: (i, 0)),
            full((_B, _DU)),
            full((_DF, _H)), full((_DF, _H)), full((_DU, _H)),
            full((_DF, _H)), full((_DF, _H)), full((_DU, _H)),
            full((1, _H)), full((1, _H)), full((1, _H)),
        ],
        out_specs=[
            pl.BlockSpec((_BN, _DF), lambda i: (i, 0)),
            pl.BlockSpec((_BN, _H), lambda i: (i, 0)),
        ],
        out_shape=[
            jax.ShapeDtypeStruct((_N, _DF), _f32),
            jax.ShapeDtypeStruct((_N, _H), _f32),
        ],
    )(x, bids2, u, eW1xs, eW1xd, eW1u, n1W1x, n2W1x, n2W1u, eb1, n1b1, n2b1)


def _sc_mesh():
    return plsc.VectorSubcoreMesh(core_axis_name="c", subcore_axis_name="s")


_K = 5
_CK = _C * _K
_NOUT = _PER_W // _CK


def _gather(src, dst, table):
    @functools.partial(
        pl.kernel,
        out_type=(jax.ShapeDtypeStruct((_E, _DF), _f32),
                  jax.ShapeDtypeStruct((_E, _DF), _f32)),
        mesh=_sc_mesh(),
        scratch_types=[
            pltpu.VMEM((_K, _C), jnp.int32),
            pltpu.VMEM((_K, _C), jnp.int32),
            pltpu.VMEM((_CK, _DF), _f32),
            pltpu.VMEM((_CK, _DF), _f32),
            pltpu.SemaphoreType.DMA,
            pltpu.SemaphoreType.DMA,
        ],
    )
    def gather_k(src_hbm, dst_hbm, t_hbm, gs_hbm, gd_hbm,
                 idx_s, idx_d, row_s, row_d, sem_a, sem_b):
        wid = lax.axis_index("s") * _NC + lax.axis_index("c")
        base = wid * _PER_W

        @pl.loop(0, _NOUT)
        def _(j):
            off = base + j * _CK
            for r in range(_K):
                pltpu.sync_copy(src_hbm.at[pl.ds(off + r * _C, _C)],
                                idx_s.at[r])
                pltpu.sync_copy(dst_hbm.at[pl.ds(off + r * _C, _C)],
                                idx_d.at[r])
            copies = []
            for r in range(_K):
                copies.append(pltpu.async_copy(
                    t_hbm.at[idx_s.at[r]],
                    row_s.at[pl.ds(r * _C, _C)], sem_a))
                copies.append(pltpu.async_copy(
                    t_hbm.at[idx_d.at[r]],
                    row_d.at[pl.ds(r * _C, _C)], sem_b))
            for cp in copies:
                cp.wait()
            pltpu.sync_copy(row_s, gs_hbm.at[pl.ds(off, _CK)])
            pltpu.sync_copy(row_d, gd_hbm.at[pl.ds(off, _CK)])

    return gather_k(src, dst, table)


def _edge_body(gs_ref, gd_ref, fe_ref, eW1a_ref, eW2_ref, eb2_ref,
               n1W1e_ref, n1W2_ref, n1b2_ref, en_ref, mext_ref):
    gd = gd_ref[...]
    h = jnp.maximum(gs_ref[:, :_H] + gd[:, _H:2 * _H]
                    + _dot_h(fe_ref[...], eW1a_ref[...]), 0.0)
    en = _dot_h(h, eW2_ref[...]) + eb2_ref[...]
    mh = jnp.maximum(gd[:, 2 * _H:3 * _H] + _dot_h(en, n1W1e_ref[...]), 0.0)
    m = _dot_h(mh, n1W2_ref[...]) + n1b2_ref[...]
    en_ref[...] = en
    mext_ref[...] = jnp.concatenate(
        [m, jnp.ones((_BE, 1), _f32), jnp.zeros((_BE, 95), _f32)], axis=1)


def _edge(gsrc, gdst, fe, eW1a, eW2, eb2, n1W1e, n1W2, n1b2):
    full = lambda shp: pl.BlockSpec(shp, lambda i: (0,) * len(shp))
    return pl.pallas_call(
        _edge_body,
        grid=(_E // _BE,),
        in_specs=[
            pl.BlockSpec((_BE, _DF), lambda i: (i, 0)),
            pl.BlockSpec((_BE, _DF), lambda i: (i, 0)),
            pl.BlockSpec((_BE, _DE), lambda i: (i, 0)),
            full((_DE, _H)), full((_H, _DE)), full((1, _DE)),
            full((_DE, _H)), full((_H, _H)), full((1, _H)),
        ],
        out_specs=[
            pl.BlockSpec((_BE, _DE), lambda i: (i, 0)),
            pl.BlockSpec((_BE, _DF), lambda i: (i, 0)),
        ],
        out_shape=[
            jax.ShapeDtypeStruct((_E, _DE), _f32),
            jax.ShapeDtypeStruct((_E, _DF), _f32),
        ],
    )(gsrc, gdst, fe, eW1a, eW2, eb2, n1W1e, n1W2, n1b2)


_NP = 10240


def _scatter(src, mext, zeros):
    rows_per_sub = _NP // _NS

    @functools.partial(
        pl.kernel,
        out_type=jax.ShapeDtypeStruct((_NC, _NP, _DF), _f32),
        mesh=_sc_mesh(),
        scratch_types=[
            pltpu.VMEM((_C,), jnp.int32),
            pltpu.VMEM((_C, _DF), _f32),
            pltpu.VMEM_SHARED((_NP, _DF), _f32),
        ],
    )
    def scatter_k(src_hbm, m_hbm, z_hbm, out_hbm, idx_v, row_v, acc):
        cid = lax.axis_index("c")
        sid = lax.axis_index("s")
        r0 = sid * rows_per_sub
        pltpu.sync_copy(z_hbm, acc.at[pl.ds(r0, rows_per_sub)])
        plsc.subcore_barrier()
        wid = sid * _NC + cid
        base = wid * _PER_W

        @pl.loop(0, _NCHUNK)
        def _(j):
            off = base + j * _C
            pltpu.sync_copy(src_hbm.at[pl.ds(off, _C)], idx_v)
            pltpu.sync_copy(m_hbm.at[pl.ds(off, _C)], row_v)
            pltpu.sync_copy(row_v, acc.at[idx_v], add=True)

        plsc.subcore_barrier()
        pltpu.sync_copy(acc.at[pl.ds(r0, rows_per_sub)],
                        out_hbm.at[cid, pl.ds(r0, rows_per_sub)])

    return scatter_k(src, mext, zeros)


def _node_body(p_ref, xp2_ref, b_ref, n2W1a_ref, n2W2_ref, n2b2_ref,
               xn_ref, gs_ref, gc_ref):
    p = p_ref[...]
    aggs = p[0, :, :_H] + p[1, :, :_H]
    cnt = p[0, :, _H:_H + 1] + p[1, :, _H:_H + 1]
    agg = aggs / jnp.maximum(cnt, 1.0)
    xh = jnp.maximum(xp2_ref[...] + _dot(agg, n2W1a_ref[...]), 0.0)
    xn = _dot(xh, n2W2_ref[...]) + n2b2_ref[...]
    xn_ref[...] = xn
    iota = lax.broadcasted_iota(jnp.int32, (_BN, _B), 1)
    oh = (b_ref[...] == iota).astype(_f32)
    gs_blk = _dot_t(oh, xn)
    gc_blk = _dot_t(oh, jnp.ones((_BN, _DF), _f32))

    @pl.when(pl.program_id(0) == 0)
    def _():
        gs_ref[...] = gs_blk
        gc_ref[...] = gc_blk

    @pl.when(pl.program_id(0) != 0)
    def _():
        gs_ref[...] += gs_blk
        gc_ref[...] += gc_blk


def _node(p, xp2, bids2, n2W1a, n2W2, n2b2):
    full = lambda shp: pl.BlockSpec(shp, lambda i: (0,) * len(shp))
    return pl.pallas_call(
        _node_body,
        grid=(_N // _BN,),
        in_specs=[
            pl.BlockSpec((_NC, _BN, _DF), lambda i: (0, i, 0)),
            pl.BlockSpec((_BN, _H), lambda i: (i, 0)),
            pl.BlockSpec((_BN, 1), lambda i: (i, 0)),
            full((_H, _H)), full((_H, _DF)), full((1, _DF)),
        ],
        out_specs=[
            pl.BlockSpec((_BN, _DF), lambda i: (i, 0)),
            full((_B, _DF)),
            full((_B, _DF)),
        ],
        out_shape=[
            jax.ShapeDtypeStruct((_N, _DF), _f32),
            jax.ShapeDtypeStruct((_B, _DF), _f32),
            jax.ShapeDtypeStruct((_B, _DF), _f32),
        ],
    )(p, xp2, bids2, n2W1a, n2W2, n2b2)


def _glob_body(u_ref, gs_ref, gc_ref, gW1u_ref, gW1m_ref, gb1_ref,
               gW2_ref, gb2_ref, un_ref):
    mean = gs_ref[...] / jnp.maximum(gc_ref[...], 1.0)
    h = jnp.maximum(_dot(u_ref[...], gW1u_ref[...]) +
                    _dot(mean, gW1m_ref[...]) + gb1_ref[...], 0.0)
    un_ref[...] = _dot(h, gW2_ref[...]) + gb2_ref[...]


def _glob(u, gs, gc, gW1u, gW1m, gb1, gW2, gb2):
    return pl.pallas_call(
        _glob_body,
        out_shape=jax.ShapeDtypeStruct((_B, _DU), _f32),
    )(u, gs, gc, gW1u, gW1m, gb1, gW2, gb2)


def kernel(features_of_nodes, node_type_ids, node_ids_for_edges,
           features_of_edges, edge_type_ids, global_features, batch_ids,
           eW1, eb1, eW2, eb2,
           n1W1, n1b1, n1W2, n1b2,
           n2W1, n2b1, n2W2, n2b2,
           gW1, gb1, gW2, gb2):
    x = features_of_nodes
    u = global_features
    src = node_ids_for_edges[0].astype(jnp.int32)
    dst = node_ids_for_edges[1].astype(jnp.int32)
    bids2 = batch_ids.astype(jnp.int32).reshape(_N, 1)

    r1 = lambda v: v.reshape(1, -1)
    table, xp2 = _precompute(
        x, bids2, u,
        eW1[:_DF], eW1[_DF:2 * _DF], eW1[2 * _DF + _DE:],
        n1W1[:_DF], n2W1[:_DF], n2W1[_DF + _H:],
        r1(eb1), r1(n1b1), r1(n2b1))

    gsrc, gdst = _gather(src, dst, table)

    e_new, mext = _edge(gsrc, gdst, features_of_edges,
                        eW1[2 * _DF:2 * _DF + _DE], eW2, r1(eb2),
                        n1W1[_DF:], n1W2, r1(n1b2))

    p = _scatter(src, mext, jnp.zeros((_NP // _NS, _DF), _f32))

    x_new, gs, gc = _node(p, xp2, bids2, n2W1[_DF:_DF + _H], n2W2, r1(n2b2))

    u_new = _glob(u, gs, gc, gW1[:_DU], gW1[_DU:], r1(gb1), gW2, r1(gb2))

    return (x_new, e_new, u_new)


# speedup vs baseline: 6.0545x; 1.0558x over previous
import functools

import jax
import jax.numpy as jnp
from jax import lax
from jax.experimental import pallas as pl
from jax.experimental.pallas import tpu as pltpu
from jax.experimental.pallas import tpu_sc as plsc

_N = 10000
_E = 320000
_B = 16
_DF = 128
_DE = 16
_DU = 32
_H = 32

_NC = 2
_NS = 16
_NW = _NC * _NS
_C = 80
_PER_W = _E // _NW
_NCHUNK = _PER_W // _C

_BN = 2000
_BE = 2560

_f32 = jnp.float32


def _dot(a, b, precision=lax.Precision.HIGHEST):
    return lax.dot_general(a, b, (((1,), (0,)), ((), ())),
                           preferred_element_type=_f32,
                           precision=precision)


def _dot_h(a, b):
    return _dot(a, b, precision=lax.Precision.DEFAULT)


def _dot_t(a, b):
    return lax.dot_general(a, b, (((0,), (0,)), ((), ())),
                           preferred_element_type=_f32,
                           precision=lax.Precision.HIGHEST)


def _pre_body(x_ref, b_ref, u_ref, eW1xs_ref, eW1xd_ref, eW1u_ref, n1W1x_ref,
              n2W1x_ref, n2W1u_ref, eb1_ref, n1b1_ref, n2b1_ref,
              tsrc_ref, xp2_ref):
    xb = x_ref[...]
    bid = b_ref[...]
    iota = lax.broadcasted_iota(jnp.int32, (_BN, _B), 1)
    oh = (bid == iota).astype(_f32)
    ue = _dot(u_ref[...], eW1u_ref[...])
    un = _dot(u_ref[...], n2W1u_ref[...])
    tsrc = _dot(xb, eW1xs_ref[...]) + _dot(oh, ue) + eb1_ref[...]
    tsrc_ref[...] = jnp.concatenate(
        [tsrc,
         _dot(xb, eW1xd_ref[...]),
         _dot(xb, n1W1x_ref[...]) + n1b1_ref[...],
         jnp.zeros((_BN, _H), _f32)], axis=1)
    xp2_ref[...] = _dot(xb, n2W1x_ref[...]) + _dot(oh, un) + n2b1_ref[...]


def _precompute(x, bids2, u, eW1xs, eW1xd, eW1u, n1W1x, n2W1x, n2W1u,
                eb1, n1b1, n2b1):
    full = lambda shp: pl.BlockSpec(shp, lambda i: (0,) * len(shp))
    return pl.pallas_call(
        _pre_body,
        grid=(_N // _BN,),
        in_specs=[
            pl.BlockSpec((_BN, _DF), lambda i: (i, 0)),
            pl.BlockSpec((_BN, 1), lambda i: (i, 0)),
            full((_B, _DU)),
            full((_DF, _H)), full((_DF, _H)), full((_DU, _H)),
            full((_DF, _H)), full((_DF, _H)), full((_DU, _H)),
            full((1, _H)), full((1, _H)), full((1, _H)),
        ],
        out_specs=[
            pl.BlockSpec((_BN, _DF), lambda i: (i, 0)),
            pl.BlockSpec((_BN, _H), lambda i: (i, 0)),
        ],
        out_shape=[
            jax.ShapeDtypeStruct((_N, _DF), _f32),
            jax.ShapeDtypeStruct((_N, _H), _f32),
        ],
    )(x, bids2, u, eW1xs, eW1xd, eW1u, n1W1x, n2W1x, n2W1u, eb1, n1b1, n2b1)


def _sc_mesh():
    return plsc.VectorSubcoreMesh(core_axis_name="c", subcore_axis_name="s")


def _gather(src, dst, table, es):
    pw = es // _NW
    nch = pw // _C

    @functools.partial(
        pl.kernel,
        out_type=(jax.ShapeDtypeStruct((es, _DF), _f32),
                  jax.ShapeDtypeStruct((es, _DF), _f32)),
        mesh=_sc_mesh(),
        scratch_types=[
            pltpu.VMEM((_C,), jnp.int32),
            pltpu.VMEM((_C,), jnp.int32),
            pltpu.VMEM((_C, _DF), _f32),
            pltpu.VMEM((_C, _DF), _f32),
            pltpu.SemaphoreType.DMA,
            pltpu.SemaphoreType.DMA,
        ],
    )
    def gather_k(src_hbm, dst_hbm, t_hbm, gs_hbm, gd_hbm,
                 idx_s, idx_d, row_s, row_d, sem_a, sem_b):
        wid = lax.axis_index("s") * _NC + lax.axis_index("c")
        base = wid * pw

        @pl.loop(0, nch)
        def _(j):
            off = base + j * _C
            pltpu.sync_copy(src_hbm.at[pl.ds(off, _C)], idx_s)
            pltpu.sync_copy(dst_hbm.at[pl.ds(off, _C)], idx_d)
            cp_a = pltpu.async_copy(t_hbm.at[idx_s], row_s, sem_a)
            cp_b = pltpu.async_copy(t_hbm.at[idx_d], row_d, sem_b)
            cp_a.wait()
            cp_b.wait()
            pltpu.sync_copy(row_s, gs_hbm.at[pl.ds(off, _C)])
            pltpu.sync_copy(row_d, gd_hbm.at[pl.ds(off, _C)])

    return gather_k(src, dst, table)


def _edge_body(gs_ref, gd_ref, fe_ref, eW1a_ref, eW2_ref, eb2_ref,
               n1W1e_ref, n1W2_ref, n1b2_ref, en_ref, mext_ref):
    gd = gd_ref[...]
    h = jnp.maximum(gs_ref[:, :_H] + gd[:, _H:2 * _H]
                    + _dot_h(fe_ref[...], eW1a_ref[...]), 0.0)
    en = _dot_h(h, eW2_ref[...]) + eb2_ref[...]
    mh = jnp.maximum(gd[:, 2 * _H:3 * _H] + _dot_h(en, n1W1e_ref[...]), 0.0)
    m = _dot_h(mh, n1W2_ref[...]) + n1b2_ref[...]
    en_ref[...] = en
    mext_ref[...] = jnp.concatenate(
        [m, jnp.ones((_BE, 1), _f32), jnp.zeros((_BE, 95), _f32)], axis=1)


def _edge(gsrc, gdst, fe, eW1a, eW2, eb2, n1W1e, n1W2, n1b2, es):
    full = lambda shp: pl.BlockSpec(shp, lambda i: (0,) * len(shp))
    return pl.pallas_call(
        _edge_body,
        grid=(es // _BE,),
        in_specs=[
            pl.BlockSpec((_BE, _DF), lambda i: (i, 0)),
            pl.BlockSpec((_BE, _DF), lambda i: (i, 0)),
            pl.BlockSpec((_BE, _DE), lambda i: (i, 0)),
            full((_DE, _H)), full((_H, _DE)), full((1, _DE)),
            full((_DE, _H)), full((_H, _H)), full((1, _H)),
        ],
        out_specs=[
            pl.BlockSpec((_BE, _DE), lambda i: (i, 0)),
            pl.BlockSpec((_BE, _DF), lambda i: (i, 0)),
        ],
        out_shape=[
            jax.ShapeDtypeStruct((es, _DE), _f32),
            jax.ShapeDtypeStruct((es, _DF), _f32),
        ],
    )(gsrc, gdst, fe, eW1a, eW2, eb2, n1W1e, n1W2, n1b2)


_NP = 10240


def _scatter(src, mext, zeros, es):
    rows_per_sub = _NP // _NS
    pw = es // _NW
    nch = pw // _C

    @functools.partial(
        pl.kernel,
        out_type=jax.ShapeDtypeStruct((_NC, _NP, _DF), _f32),
        mesh=_sc_mesh(),
        scratch_types=[
            pltpu.VMEM((_C,), jnp.int32),
            pltpu.VMEM((_C, _DF), _f32),
            pltpu.VMEM_SHARED((_NP, _DF), _f32),
        ],
    )
    def scatter_k(src_hbm, m_hbm, z_hbm, out_hbm, idx_v, row_v, acc):
        cid = lax.axis_index("c")
        sid = lax.axis_index("s")
        r0 = sid * rows_per_sub
        pltpu.sync_copy(z_hbm, acc.at[pl.ds(r0, rows_per_sub)])
        plsc.subcore_barrier()
        wid = sid * _NC + cid
        base = wid * pw

        @pl.loop(0, nch)
        def _(j):
            off = base + j * _C
            pltpu.sync_copy(src_hbm.at[pl.ds(off, _C)], idx_v)
            pltpu.sync_copy(m_hbm.at[pl.ds(off, _C)], row_v)
            pltpu.sync_copy(row_v, acc.at[idx_v], add=True)

        plsc.subcore_barrier()
        pltpu.sync_copy(acc.at[pl.ds(r0, rows_per_sub)],
                        out_hbm.at[cid, pl.ds(r0, rows_per_sub)])

    return scatter_k(src, mext, zeros)


_EA = 161280


def _node_body(pa_ref, pb_ref, xp2_ref, b_ref, n2W1a_ref, n2W2_ref, n2b2_ref,
               xn_ref, gs_ref, gc_ref):
    p = pa_ref[...] + pb_ref[...]
    aggs = p[0, :, :_H] + p[1, :, :_H]
    cnt = p[0, :, _H:_H + 1] + p[1, :, _H:_H + 1]
    agg = aggs / jnp.maximum(cnt, 1.0)
    xh = jnp.maximum(xp2_ref[...] + _dot(agg, n2W1a_ref[...]), 0.0)
    xn = _dot(xh, n2W2_ref[...]) + n2b2_ref[...]
    xn_ref[...] = xn
    iota = lax.broadcasted_iota(jnp.int32, (_BN, _B), 1)
    oh = (b_ref[...] == iota).astype(_f32)
    gs_blk = _dot_t(oh, xn)
    gc_blk = _dot_t(oh, jnp.ones((_BN, _DF), _f32))

    @pl.when(pl.program_id(0) == 0)
    def _():
        gs_ref[...] = gs_blk
        gc_ref[...] = gc_blk

    @pl.when(pl.program_id(0) != 0)
    def _():
        gs_ref[...] += gs_blk
        gc_ref[...] += gc_blk


def _node(pa, pb, xp2, bids2, n2W1a, n2W2, n2b2):
    full = lambda shp: pl.BlockSpec(shp, lambda i: (0,) * len(shp))
    return pl.pallas_call(
        _node_body,
        grid=(_N // _BN,),
        in_specs=[
            pl.BlockSpec((_NC, _BN, _DF), lambda i: (0, i, 0)),
            pl.BlockSpec((_NC, _BN, _DF), lambda i: (0, i, 0)),
            pl.BlockSpec((_BN, _H), lambda i: (i, 0)),
            pl.BlockSpec((_BN, 1), lambda i: (i, 0)),
            full((_H, _H)), full((_H, _DF)), full((1, _DF)),
        ],
        out_specs=[
            pl.BlockSpec((_BN, _DF), lambda i: (i, 0)),
            full((_B, _DF)),
            full((_B, _DF)),
        ],
        out_shape=[
            jax.ShapeDtypeStruct((_N, _DF), _f32),
            jax.ShapeDtypeStruct((_B, _DF), _f32),
            jax.ShapeDtypeStruct((_B, _DF), _f32),
        ],
    )(pa, pb, xp2, bids2, n2W1a, n2W2, n2b2)


def _glob_body(u_ref, gs_ref, gc_ref, gW1u_ref, gW1m_ref, gb1_ref,
               gW2_ref, gb2_ref, un_ref):
    mean = gs_ref[...] / jnp.maximum(gc_ref[...], 1.0)
    h = jnp.maximum(_dot(u_ref[...], gW1u_ref[...]) +
                    _dot(mean, gW1m_ref[...]) + gb1_ref[...], 0.0)
    un_ref[...] = _dot(h, gW2_ref[...]) + gb2_ref[...]


def _glob(u, gs, gc, gW1u, gW1m, gb1, gW2, gb2):
    return pl.pallas_call(
        _glob_body,
        out_shape=jax.ShapeDtypeStruct((_B, _DU), _f32),
    )(u, gs, gc, gW1u, gW1m, gb1, gW2, gb2)


def kernel(features_of_nodes, node_type_ids, node_ids_for_edges,
           features_of_edges, edge_type_ids, global_features, batch_ids,
           eW1, eb1, eW2, eb2,
           n1W1, n1b1, n1W2, n1b2,
           n2W1, n2b1, n2W2, n2b2,
           gW1, gb1, gW2, gb2):
    x = features_of_nodes
    u = global_features
    src = node_ids_for_edges[0].astype(jnp.int32)
    dst = node_ids_for_edges[1].astype(jnp.int32)
    bids2 = batch_ids.astype(jnp.int32).reshape(_N, 1)

    r1 = lambda v: v.reshape(1, -1)
    table, xp2 = _precompute(
        x, bids2, u,
        eW1[:_DF], eW1[_DF:2 * _DF], eW1[2 * _DF + _DE:],
        n1W1[:_DF], n2W1[:_DF], n2W1[_DF + _H:],
        r1(eb1), r1(n1b1), r1(n2b1))

    zrows = jnp.zeros((_NP // _NS, _DF), _f32)
    eb = _E - _EA
    src_a, src_b = src[:_EA], src[_EA:]
    gs_a, gd_a = _gather(src_a, dst[:_EA], table, _EA)
    gs_b, gd_b = _gather(src_b, dst[_EA:], table, eb)

    ew = (eW1[2 * _DF:2 * _DF + _DE], eW2, r1(eb2), n1W1[_DF:], n1W2,
          r1(n1b2))
    en_a, mx_a = _edge(gs_a, gd_a, features_of_edges[:_EA], *ew, _EA)
    en_b, mx_b = _edge(gs_b, gd_b, features_of_edges[_EA:], *ew, eb)

    p_a = _scatter(src_a, mx_a, zrows, _EA)
    p_b = _scatter(src_b, mx_b, zrows, eb)
    e_new = jnp.concatenate([en_a, en_b], axis=0)

    x_new, gs, gc = _node(p_a, p_b, xp2, bids2, n2W1[_DF:_DF + _H], n2W2,
                          r1(n2b2))

    u_new = _glob(u, gs, gc, gW1[:_DU], gW1[_DU:], r1(gb1), gW2, r1(gb2))

    return (x_new, e_new, u_new)


# speedup vs baseline: 6.3502x; 1.0488x over previous
import functools

import jax
import jax.numpy as jnp
from jax import lax
from jax.experimental import pallas as pl
from jax.experimental.pallas import tpu as pltpu
from jax.experimental.pallas import tpu_sc as plsc

_N = 10000
_E = 320000
_B = 16
_DF = 128
_DE = 16
_DU = 32
_H = 32

_NC = 2
_NS = 16
_NW = _NC * _NS
_C = 80
_K = 5
_PER_W = _E // _NW
_NCHUNK = _PER_W // _C

_BN = 2000
_BE = 2560

_f32 = jnp.float32


def _dot(a, b, precision=lax.Precision.HIGHEST):
    return lax.dot_general(a, b, (((1,), (0,)), ((), ())),
                           preferred_element_type=_f32,
                           precision=precision)


def _dot_h(a, b):
    return _dot(a, b, precision=lax.Precision.DEFAULT)


def _dot_t(a, b):
    return lax.dot_general(a, b, (((0,), (0,)), ((), ())),
                           preferred_element_type=_f32,
                           precision=lax.Precision.HIGHEST)


def _pre_body(x_ref, b_ref, u_ref, eW1xs_ref, eW1xd_ref, eW1u_ref, n1W1x_ref,
              n2W1x_ref, n2W1u_ref, eb1_ref, n1b1_ref, n2b1_ref,
              tsrc_ref, xp2_ref):
    xb = x_ref[...]
    bid = b_ref[...]
    iota = lax.broadcasted_iota(jnp.int32, (_BN, _B), 1)
    oh = (bid == iota).astype(_f32)
    ue = _dot(u_ref[...], eW1u_ref[...])
    un = _dot(u_ref[...], n2W1u_ref[...])
    tsrc = _dot(xb, eW1xs_ref[...]) + _dot(oh, ue) + eb1_ref[...]
    tsrc_ref[...] = jnp.concatenate(
        [tsrc,
         _dot(xb, eW1xd_ref[...]),
         _dot(xb, n1W1x_ref[...]) + n1b1_ref[...],
         jnp.zeros((_BN, _H), _f32)], axis=1)
    xp2_ref[...] = _dot(xb, n2W1x_ref[...]) + _dot(oh, un) + n2b1_ref[...]


def _precompute(x, bids2, u, eW1xs, eW1xd, eW1u, n1W1x, n2W1x, n2W1u,
                eb1, n1b1, n2b1):
    full = lambda shp: pl.BlockSpec(shp, lambda i: (0,) * len(shp))
    return pl.pallas_call(
        _pre_body,
        grid=(_N // _BN,),
        in_specs=[
            pl.BlockSpec((_BN, _DF), lambda i: (i, 0)),
            pl.BlockSpec((_BN, 1), lambda i: (i, 0)),
            full((_B, _DU)),
            full((_DF, _H)), full((_DF, _H)), full((_DU, _H)),
            full((_DF, _H)), full((_DF, _H)), full((_DU, _H)),
            full((1, _H)), full((1, _H)), full((1, _H)),
        ],
        out_specs=[
            pl.BlockSpec((_BN, _DF), lambda i: (i, 0)),
            pl.BlockSpec((_BN, _H), lambda i: (i, 0)),
        ],
        out_shape=[
            jax.ShapeDtypeStruct((_N, _DF), _f32),
            jax.ShapeDtypeStruct((_N, _H), _f32),
        ],
    )(x, bids2, u, eW1xs, eW1xd, eW1u, n1W1x, n2W1x, n2W1u, eb1, n1b1, n2b1)


def _sc_mesh():
    return plsc.VectorSubcoreMesh(core_axis_name="c", subcore_axis_name="s")


def _gather(src, dst, table, es):
    pw = es // _NW
    nch = pw // _C

    nb = nch // _K
    rem = nch - nb * _K

    @functools.partial(
        pl.kernel,
        out_type=(jax.ShapeDtypeStruct((es, _DF), _f32),
                  jax.ShapeDtypeStruct((es, _DF), _f32)),
        mesh=_sc_mesh(),
        scratch_types=[
            pltpu.VMEM((_K, _C), jnp.int32),
            pltpu.VMEM((_K, _C), jnp.int32),
            pltpu.VMEM((_K * _C, _DF), _f32),
            pltpu.VMEM((_K * _C, _DF), _f32),
            pltpu.SemaphoreType.DMA,
            pltpu.SemaphoreType.DMA,
        ],
    )
    def gather_k(src_hbm, dst_hbm, t_hbm, gs_hbm, gd_hbm,
                 idx_s, idx_d, row_s, row_d, sem_a, sem_b):
        wid = lax.axis_index("s") * _NC + lax.axis_index("c")
        base = wid * pw

        def chunk_group(off, k):
            for r in range(k):
                pltpu.sync_copy(src_hbm.at[pl.ds(off + r * _C, _C)],
                                idx_s.at[r])
                pltpu.sync_copy(dst_hbm.at[pl.ds(off + r * _C, _C)],
                                idx_d.at[r])
            copies = []
            for r in range(k):
                copies.append(pltpu.async_copy(
                    t_hbm.at[idx_s.at[r]],
                    row_s.at[pl.ds(r * _C, _C)], sem_a))
                copies.append(pltpu.async_copy(
                    t_hbm.at[idx_d.at[r]],
                    row_d.at[pl.ds(r * _C, _C)], sem_b))
            for cp in copies:
                cp.wait()
            pltpu.sync_copy(row_s.at[pl.ds(0, k * _C)],
                            gs_hbm.at[pl.ds(off, k * _C)])
            pltpu.sync_copy(row_d.at[pl.ds(0, k * _C)],
                            gd_hbm.at[pl.ds(off, k * _C)])

        @pl.loop(0, nb)
        def _(j):
            chunk_group(base + j * (_K * _C), _K)

        if rem:
            chunk_group(base + nb * (_K * _C), rem)

    return gather_k(src, dst, table)


def _edge_body(gs_ref, gd_ref, fe_ref, eW1a_ref, eW2_ref, eb2_ref,
               n1W1e_ref, n1W2_ref, n1b2_ref, en_ref, mext_ref):
    gs = gs_ref[...]
    gd = gd_ref[...]
    h = jnp.maximum(gs[:, :_H] + gd[:, _H:2 * _H]
                    + _dot_h(fe_ref[...], eW1a_ref[...]), 0.0)
    en = _dot_h(h, eW2_ref[...]) + eb2_ref[...]
    mh = jnp.maximum(gd[:, 2 * _H:3 * _H] + _dot_h(en, n1W1e_ref[...]), 0.0)
    m = _dot_h(mh, n1W2_ref[...]) + n1b2_ref[...]
    en_ref[...] = en
    mext_ref[...] = jnp.concatenate(
        [m, jnp.ones((_BE, 1), _f32), jnp.zeros((_BE, 95), _f32)], axis=1)


def _edge(gs, gd, fe, eW1a, eW2, eb2, n1W1e, n1W2, n1b2, es):
    full = lambda shp: pl.BlockSpec(shp, lambda i: (0,) * len(shp))
    return pl.pallas_call(
        _edge_body,
        grid=(es // _BE,),
        in_specs=[
            pl.BlockSpec((_BE, _DF), lambda i: (i, 0)),
            pl.BlockSpec((_BE, _DF), lambda i: (i, 0)),
            pl.BlockSpec((_BE, _DE), lambda i: (i, 0)),
            full((_DE, _H)), full((_H, _DE)), full((1, _DE)),
            full((_DE, _H)), full((_H, _H)), full((1, _H)),
        ],
        out_specs=[
            pl.BlockSpec((_BE, _DE), lambda i: (i, 0)),
            pl.BlockSpec((_BE, _DF), lambda i: (i, 0)),
        ],
        out_shape=[
            jax.ShapeDtypeStruct((es, _DE), _f32),
            jax.ShapeDtypeStruct((es, _DF), _f32),
        ],
    )(gs, gd, fe, eW1a, eW2, eb2, n1W1e, n1W2, n1b2)


_NP = 10240


def _scatter(src, mext, zeros, es):
    rows_per_sub = _NP // _NS
    pw = es // _NW
    nch = pw // _C

    @functools.partial(
        pl.kernel,
        out_type=jax.ShapeDtypeStruct((_NC, _NP, _DF), _f32),
        mesh=_sc_mesh(),
        scratch_types=[
            pltpu.VMEM((_C,), jnp.int32),
            pltpu.VMEM((_C, _DF), _f32),
            pltpu.VMEM_SHARED((_NP, _DF), _f32),
        ],
    )
    def scatter_k(src_hbm, m_hbm, z_hbm, out_hbm, idx_v, row_v, acc):
        cid = lax.axis_index("c")
        sid = lax.axis_index("s")
        r0 = sid * rows_per_sub
        pltpu.sync_copy(z_hbm, acc.at[pl.ds(r0, rows_per_sub)])
        plsc.subcore_barrier()
        wid = sid * _NC + cid
        base = wid * pw

        @pl.loop(0, nch)
        def _(j):
            off = base + j * _C
            pltpu.sync_copy(src_hbm.at[pl.ds(off, _C)], idx_v)
            pltpu.sync_copy(m_hbm.at[pl.ds(off, _C)], row_v)
            pltpu.sync_copy(row_v, acc.at[idx_v], add=True)

        plsc.subcore_barrier()
        pltpu.sync_copy(acc.at[pl.ds(r0, rows_per_sub)],
                        out_hbm.at[cid, pl.ds(r0, rows_per_sub)])

    return scatter_k(src, mext, zeros)


_EA = 161280


def _node_body(pa_ref, pb_ref, xp2_ref, b_ref, n2W1a_ref, n2W2_ref, n2b2_ref,
               xn_ref, gs_ref, gc_ref):
    p = pa_ref[...] + pb_ref[...]
    aggs = p[0, :, :_H] + p[1, :, :_H]
    cnt = p[0, :, _H:_H + 1] + p[1, :, _H:_H + 1]
    agg = aggs / jnp.maximum(cnt, 1.0)
    xh = jnp.maximum(xp2_ref[...] + _dot(agg, n2W1a_ref[...]), 0.0)
    xn = _dot(xh, n2W2_ref[...]) + n2b2_ref[...]
    xn_ref[...] = xn
    iota = lax.broadcasted_iota(jnp.int32, (_BN, _B), 1)
    oh = (b_ref[...] == iota).astype(_f32)
    gs_blk = _dot_t(oh, xn)
    gc_blk = _dot_t(oh, jnp.ones((_BN, _DF), _f32))

    @pl.when(pl.program_id(0) == 0)
    def _():
        gs_ref[...] = gs_blk
        gc_ref[...] = gc_blk

    @pl.when(pl.program_id(0) != 0)
    def _():
        gs_ref[...] += gs_blk
        gc_ref[...] += gc_blk


def _node(pa, pb, xp2, bids2, n2W1a, n2W2, n2b2):
    full = lambda shp: pl.BlockSpec(shp, lambda i: (0,) * len(shp))
    return pl.pallas_call(
        _node_body,
        grid=(_N // _BN,),
        in_specs=[
            pl.BlockSpec((_NC, _BN, _DF), lambda i: (0, i, 0)),
            pl.BlockSpec((_NC, _BN, _DF), lambda i: (0, i, 0)),
            pl.BlockSpec((_BN, _H), lambda i: (i, 0)),
            pl.BlockSpec((_BN, 1), lambda i: (i, 0)),
            full((_H, _H)), full((_H, _DF)), full((1, _DF)),
        ],
        out_specs=[
            pl.BlockSpec((_BN, _DF), lambda i: (i, 0)),
            full((_B, _DF)),
            full((_B, _DF)),
        ],
        out_shape=[
            jax.ShapeDtypeStruct((_N, _DF), _f32),
            jax.ShapeDtypeStruct((_B, _DF), _f32),
            jax.ShapeDtypeStruct((_B, _DF), _f32),
        ],
    )(pa, pb, xp2, bids2, n2W1a, n2W2, n2b2)


def _glob_body(u_ref, gs_ref, gc_ref, gW1u_ref, gW1m_ref, gb1_ref,
               gW2_ref, gb2_ref, un_ref):
    mean = gs_ref[...] / jnp.maximum(gc_ref[...], 1.0)
    h = jnp.maximum(_dot(u_ref[...], gW1u_ref[...]) +
                    _dot(mean, gW1m_ref[...]) + gb1_ref[...], 0.0)
    un_ref[...] = _dot(h, gW2_ref[...]) + gb2_ref[...]


def _glob(u, gs, gc, gW1u, gW1m, gb1, gW2, gb2):
    return pl.pallas_call(
        _glob_body,
        out_shape=jax.ShapeDtypeStruct((_B, _DU), _f32),
    )(u, gs, gc, gW1u, gW1m, gb1, gW2, gb2)


def kernel(features_of_nodes, node_type_ids, node_ids_for_edges,
           features_of_edges, edge_type_ids, global_features, batch_ids,
           eW1, eb1, eW2, eb2,
           n1W1, n1b1, n1W2, n1b2,
           n2W1, n2b1, n2W2, n2b2,
           gW1, gb1, gW2, gb2):
    x = features_of_nodes
    u = global_features
    src = node_ids_for_edges[0].astype(jnp.int32)
    dst = node_ids_for_edges[1].astype(jnp.int32)
    bids2 = batch_ids.astype(jnp.int32).reshape(_N, 1)

    r1 = lambda v: v.reshape(1, -1)
    table, xp2 = _precompute(
        x, bids2, u,
        eW1[:_DF], eW1[_DF:2 * _DF], eW1[2 * _DF + _DE:],
        n1W1[:_DF], n2W1[:_DF], n2W1[_DF + _H:],
        r1(eb1), r1(n1b1), r1(n2b1))

    zrows = jnp.zeros((_NP // _NS, _DF), _f32)
    eb = _E - _EA
    src_a, src_b = src[:_EA], src[_EA:]
    gs_a, gd_a = _gather(src_a, dst[:_EA], table, _EA)
    gs_b, gd_b = _gather(src_b, dst[_EA:], table, eb)

    ew = (eW1[2 * _DF:2 * _DF + _DE], eW2, r1(eb2), n1W1[_DF:], n1W2,
          r1(n1b2))
    en_a, mx_a = _edge(gs_a, gd_a, features_of_edges[:_EA], *ew, _EA)
    en_b, mx_b = _edge(gs_b, gd_b, features_of_edges[_EA:], *ew, eb)

    p_a = _scatter(src_a, mx_a, zrows, _EA)
    p_b = _scatter(src_b, mx_b, zrows, eb)
    e_new = jnp.concatenate([en_a, en_b], axis=0)

    x_new, gs, gc = _node(p_a, p_b, xp2, bids2, n2W1[_DF:_DF + _H], n2W2,
                          r1(n2b2))

    u_new = _glob(u, gs, gc, gW1[:_DU], gW1[_DU:], r1(gb1), gW2, r1(gb2))

    return (x_new, e_new, u_new)


# speedup vs baseline: 6.5660x; 1.0340x over previous
import functools

import jax
import jax.numpy as jnp
from jax import lax
from jax.experimental import pallas as pl
from jax.experimental.pallas import tpu as pltpu
from jax.experimental.pallas import tpu_sc as plsc

_N = 10000
_E = 320000
_B = 16
_DF = 128
_DE = 16
_DU = 32
_H = 32

_NC = 2
_NS = 16
_NW = _NC * _NS
_C = 80
_K = 5
_PER_W = _E // _NW
_NCHUNK = _PER_W // _C

_BN = 2000
_BE = 2560

_f32 = jnp.float32


def _dot(a, b, precision=lax.Precision.HIGHEST):
    return lax.dot_general(a, b, (((1,), (0,)), ((), ())),
                           preferred_element_type=_f32,
                           precision=precision)


def _dot_h(a, b):
    return _dot(a, b, precision=lax.Precision.DEFAULT)


def _dot_t(a, b):
    return lax.dot_general(a, b, (((0,), (0,)), ((), ())),
                           preferred_element_type=_f32,
                           precision=lax.Precision.HIGHEST)


def _pre_body(x_ref, b_ref, u_ref, eW1xs_ref, eW1xd_ref, eW1u_ref, n1W1x_ref,
              n2W1x_ref, n2W1u_ref, eb1_ref, n1b1_ref, n2b1_ref,
              tsrc_ref, xp2_ref):
    xb = x_ref[...]
    bid = b_ref[...]
    iota = lax.broadcasted_iota(jnp.int32, (_BN, _B), 1)
    oh = (bid == iota).astype(_f32)
    ue = _dot(u_ref[...], eW1u_ref[...])
    un = _dot(u_ref[...], n2W1u_ref[...])
    tsrc = _dot(xb, eW1xs_ref[...]) + _dot(oh, ue) + eb1_ref[...]
    tsrc_ref[...] = jnp.concatenate(
        [tsrc,
         _dot(xb, eW1xd_ref[...]),
         _dot(xb, n1W1x_ref[...]) + n1b1_ref[...],
         jnp.zeros((_BN, _H), _f32)], axis=1)
    xp2_ref[...] = _dot(xb, n2W1x_ref[...]) + _dot(oh, un) + n2b1_ref[...]


def _precompute(x, bids2, u, eW1xs, eW1xd, eW1u, n1W1x, n2W1x, n2W1u,
                eb1, n1b1, n2b1):
    full = lambda shp: pl.BlockSpec(shp, lambda i: (0,) * len(shp))
    return pl.pallas_call(
        _pre_body,
        grid=(_N // _BN,),
        in_specs=[
            pl.BlockSpec((_BN, _DF), lambda i: (i, 0)),
            pl.BlockSpec((_BN, 1), lambda i: (i, 0)),
            full((_B, _DU)),
            full((_DF, _H)), full((_DF, _H)), full((_DU, _H)),
            full((_DF, _H)), full((_DF, _H)), full((_DU, _H)),
            full((1, _H)), full((1, _H)), full((1, _H)),
        ],
        out_specs=[
            pl.BlockSpec((_BN, _DF), lambda i: (i, 0)),
            pl.BlockSpec((_BN, _H), lambda i: (i, 0)),
        ],
        out_shape=[
            jax.ShapeDtypeStruct((_N, _DF), _f32),
            jax.ShapeDtypeStruct((_N, _H), _f32),
        ],
    )(x, bids2, u, eW1xs, eW1xd, eW1u, n1W1x, n2W1x, n2W1u, eb1, n1b1, n2b1)


def _sc_mesh():
    return plsc.VectorSubcoreMesh(core_axis_name="c", subcore_axis_name="s")


def _gather(src, dst, table, es, c=_C, k=_K):
    pw = es // _NW
    nch = pw // c

    nb = nch // k
    rem = nch - nb * k

    @functools.partial(
        pl.kernel,
        out_type=(jax.ShapeDtypeStruct((es, _DF), _f32),
                  jax.ShapeDtypeStruct((es, _DF), _f32)),
        mesh=_sc_mesh(),
        scratch_types=[
            pltpu.VMEM((k, c), jnp.int32),
            pltpu.VMEM((k, c), jnp.int32),
            pltpu.VMEM((k * c, _DF), _f32),
            pltpu.VMEM((k * c, _DF), _f32),
            pltpu.SemaphoreType.DMA,
            pltpu.SemaphoreType.DMA,
        ],
    )
    def gather_k(src_hbm, dst_hbm, t_hbm, gs_hbm, gd_hbm,
                 idx_s, idx_d, row_s, row_d, sem_a, sem_b):
        wid = lax.axis_index("s") * _NC + lax.axis_index("c")
        base = wid * pw

        def chunk_group(off, k):
            for r in range(k):
                pltpu.sync_copy(src_hbm.at[pl.ds(off + r * c, c)],
                                idx_s.at[r])
                pltpu.sync_copy(dst_hbm.at[pl.ds(off + r * c, c)],
                                idx_d.at[r])
            copies = []
            for r in range(k):
                copies.append(pltpu.async_copy(
                    t_hbm.at[idx_s.at[r]],
                    row_s.at[pl.ds(r * c, c)], sem_a))
                copies.append(pltpu.async_copy(
                    t_hbm.at[idx_d.at[r]],
                    row_d.at[pl.ds(r * c, c)], sem_b))
            for cp in copies:
                cp.wait()
            pltpu.sync_copy(row_s.at[pl.ds(0, k * c)],
                            gs_hbm.at[pl.ds(off, k * c)])
            pltpu.sync_copy(row_d.at[pl.ds(0, k * c)],
                            gd_hbm.at[pl.ds(off, k * c)])

        @pl.loop(0, nb)
        def _(j):
            chunk_group(base + j * (k * c), k)

        if rem:
            chunk_group(base + nb * (k * c), rem)

    return gather_k(src, dst, table)


def _edge_body(gs_ref, gd_ref, fe_ref, eW1a_ref, eW2_ref, eb2_ref,
               n1W1e_ref, n1W2_ref, n1b2_ref, en_ref, mext_ref):
    gs = gs_ref[...]
    gd = gd_ref[...]
    h = jnp.maximum(gs[:, :_H] + gd[:, _H:2 * _H]
                    + _dot_h(fe_ref[...], eW1a_ref[...]), 0.0)
    en = _dot_h(h, eW2_ref[...]) + eb2_ref[...]
    mh = jnp.maximum(gd[:, 2 * _H:3 * _H] + _dot_h(en, n1W1e_ref[...]), 0.0)
    m = _dot_h(mh, n1W2_ref[...]) + n1b2_ref[...]
    en_ref[...] = en
    mext_ref[...] = jnp.concatenate(
        [m, jnp.ones((_BE, 1), _f32), jnp.zeros((_BE, 95), _f32)], axis=1)


def _edge(gs, gd, fe, eW1a, eW2, eb2, n1W1e, n1W2, n1b2, es):
    full = lambda shp: pl.BlockSpec(shp, lambda i: (0,) * len(shp))
    return pl.pallas_call(
        _edge_body,
        grid=(es // _BE,),
        in_specs=[
            pl.BlockSpec((_BE, _DF), lambda i: (i, 0)),
            pl.BlockSpec((_BE, _DF), lambda i: (i, 0)),
            pl.BlockSpec((_BE, _DE), lambda i: (i, 0)),
            full((_DE, _H)), full((_H, _DE)), full((1, _DE)),
            full((_DE, _H)), full((_H, _H)), full((1, _H)),
        ],
        out_specs=[
            pl.BlockSpec((_BE, _DE), lambda i: (i, 0)),
            pl.BlockSpec((_BE, _DF), lambda i: (i, 0)),
        ],
        out_shape=[
            jax.ShapeDtypeStruct((es, _DE), _f32),
            jax.ShapeDtypeStruct((es, _DF), _f32),
        ],
    )(gs, gd, fe, eW1a, eW2, eb2, n1W1e, n1W2, n1b2)


_NP = 10240


def _scatter(src, mext, zeros, es, c=_C):
    rows_per_sub = _NP // _NS
    pw = es // _NW
    nch = pw // c

    @functools.partial(
        pl.kernel,
        out_type=jax.ShapeDtypeStruct((_NC, _NP, _DF), _f32),
        mesh=_sc_mesh(),
        scratch_types=[
            pltpu.VMEM((c,), jnp.int32),
            pltpu.VMEM((c, _DF), _f32),
            pltpu.VMEM_SHARED((_NP, _DF), _f32),
        ],
    )
    def scatter_k(src_hbm, m_hbm, z_hbm, out_hbm, idx_v, row_v, acc):
        cid = lax.axis_index("c")
        sid = lax.axis_index("s")
        r0 = sid * rows_per_sub
        pltpu.sync_copy(z_hbm, acc.at[pl.ds(r0, rows_per_sub)])
        plsc.subcore_barrier()
        wid = sid * _NC + cid
        base = wid * pw

        @pl.loop(0, nch)
        def _(j):
            off = base + j * c
            pltpu.sync_copy(src_hbm.at[pl.ds(off, c)], idx_v)
            pltpu.sync_copy(m_hbm.at[pl.ds(off, c)], row_v)
            pltpu.sync_copy(row_v, acc.at[idx_v], add=True)

        plsc.subcore_barrier()
        pltpu.sync_copy(acc.at[pl.ds(r0, rows_per_sub)],
                        out_hbm.at[cid, pl.ds(r0, rows_per_sub)])

    return scatter_k(src, mext, zeros)


_EA = 161280


def _node_body(pa_ref, pb_ref, xp2_ref, b_ref, n2W1a_ref, n2W2_ref, n2b2_ref,
               xn_ref, gs_ref, gc_ref):
    p = pa_ref[...] + pb_ref[...]
    aggs = p[0, :, :_H] + p[1, :, :_H]
    cnt = p[0, :, _H:_H + 1] + p[1, :, _H:_H + 1]
    agg = aggs / jnp.maximum(cnt, 1.0)
    xh = jnp.maximum(xp2_ref[...] + _dot(agg, n2W1a_ref[...]), 0.0)
    xn = _dot(xh, n2W2_ref[...]) + n2b2_ref[...]
    xn_ref[...] = xn
    iota = lax.broadcasted_iota(jnp.int32, (_BN, _B), 1)
    oh = (b_ref[...] == iota).astype(_f32)
    gs_blk = _dot_t(oh, xn)
    gc_blk = _dot_t(oh, jnp.ones((_BN, _DF), _f32))

    @pl.when(pl.program_id(0) == 0)
    def _():
        gs_ref[...] = gs_blk
        gc_ref[...] = gc_blk

    @pl.when(pl.program_id(0) != 0)
    def _():
        gs_ref[...] += gs_blk
        gc_ref[...] += gc_blk


def _node(pa, pb, xp2, bids2, n2W1a, n2W2, n2b2):
    full = lambda shp: pl.BlockSpec(shp, lambda i: (0,) * len(shp))
    return pl.pallas_call(
        _node_body,
        grid=(_N // _BN,),
        in_specs=[
            pl.BlockSpec((_NC, _BN, _DF), lambda i: (0, i, 0)),
            pl.BlockSpec((_NC, _BN, _DF), lambda i: (0, i, 0)),
            pl.BlockSpec((_BN, _H), lambda i: (i, 0)),
            pl.BlockSpec((_BN, 1), lambda i: (i, 0)),
            full((_H, _H)), full((_H, _DF)), full((1, _DF)),
        ],
        out_specs=[
            pl.BlockSpec((_BN, _DF), lambda i: (i, 0)),
            full((_B, _DF)),
            full((_B, _DF)),
        ],
        out_shape=[
            jax.ShapeDtypeStruct((_N, _DF), _f32),
            jax.ShapeDtypeStruct((_B, _DF), _f32),
            jax.ShapeDtypeStruct((_B, _DF), _f32),
        ],
    )(pa, pb, xp2, bids2, n2W1a, n2W2, n2b2)


def _glob_body(u_ref, gs_ref, gc_ref, gW1u_ref, gW1m_ref, gb1_ref,
               gW2_ref, gb2_ref, un_ref):
    mean = gs_ref[...] / jnp.maximum(gc_ref[...], 1.0)
    h = jnp.maximum(_dot(u_ref[...], gW1u_ref[...]) +
                    _dot(mean, gW1m_ref[...]) + gb1_ref[...], 0.0)
    un_ref[...] = _dot(h, gW2_ref[...]) + gb2_ref[...]


def _glob(u, gs, gc, gW1u, gW1m, gb1, gW2, gb2):
    return pl.pallas_call(
        _glob_body,
        out_shape=jax.ShapeDtypeStruct((_B, _DU), _f32),
    )(u, gs, gc, gW1u, gW1m, gb1, gW2, gb2)


def kernel(features_of_nodes, node_type_ids, node_ids_for_edges,
           features_of_edges, edge_type_ids, global_features, batch_ids,
           eW1, eb1, eW2, eb2,
           n1W1, n1b1, n1W2, n1b2,
           n2W1, n2b1, n2W2, n2b2,
           gW1, gb1, gW2, gb2):
    x = features_of_nodes
    u = global_features
    src = node_ids_for_edges[0].astype(jnp.int32)
    dst = node_ids_for_edges[1].astype(jnp.int32)
    bids2 = batch_ids.astype(jnp.int32).reshape(_N, 1)

    r1 = lambda v: v.reshape(1, -1)
    table, xp2 = _precompute(
        x, bids2, u,
        eW1[:_DF], eW1[_DF:2 * _DF], eW1[2 * _DF + _DE:],
        n1W1[:_DF], n2W1[:_DF], n2W1[_DF + _H:],
        r1(eb1), r1(n1b1), r1(n2b1))

    zrows = jnp.zeros((_NP // _NS, _DF), _f32)
    eb = _E - _EA
    src_a, src_b = src[:_EA], src[_EA:]
    gs_a, gd_a = _gather(src_a, dst[:_EA], table, _EA, c=112, k=3)
    gs_b, gd_b = _gather(src_b, dst[_EA:], table, eb)

    ew = (eW1[2 * _DF:2 * _DF + _DE], eW2, r1(eb2), n1W1[_DF:], n1W2,
          r1(n1b2))
    en_a, mx_a = _edge(gs_a, gd_a, features_of_edges[:_EA], *ew, _EA)
    en_b, mx_b = _edge(gs_b, gd_b, features_of_edges[_EA:], *ew, eb)

    p_a = _scatter(src_a, mx_a, zrows, _EA, c=112)
    p_b = _scatter(src_b, mx_b, zrows, eb)
    e_new = jnp.concatenate([en_a, en_b], axis=0)

    x_new, gs, gc = _node(p_a, p_b, xp2, bids2, n2W1[_DF:_DF + _H], n2W2,
                          r1(n2b2))

    u_new = _glob(u, gs, gc, gW1[:_DU], gW1[_DU:], r1(gb1), gW2, r1(gb2))

    return (x_new, e_new, u_new)
